# Initial kernel scaffold; baseline (speedup 1.0000x reference)
import jax
import jax.numpy as jnp
from jax.experimental import pallas as pl


def kernel(x_querry, l, x_block, e_p_0, e_k_0):
    raise NotImplementedError("write your pallas kernel here")


# speedup vs baseline: 9.0196x; 9.0196x over previous
import functools

import jax
import jax.numpy as jnp
from jax import lax
from jax.experimental import pallas as pl
from jax.experimental.pallas import tpu as pltpu
from jax.experimental.pallas import tpu_sc as plsc

B = 128
KEY_D = 768
EMB_D = 768
POOL = 128
PLEN = 8
FDIM = PLEN * EMB_D
NW = 16
ROWS_PER_W = B // NW


def _score_body(x_ref, k_ref, p_ref, idx_ref, loss_ref):
    x = x_ref[...]
    K = k_ref[...]
    p = p_ref[...]
    k_norm = jnp.sqrt(jnp.sum(K * K, axis=1, keepdims=True))
    n_K = K / jnp.maximum(k_norm, 1e-12)
    cos = lax.dot_general(x, n_K, (((1,), (1,)), ((), ())),
                          preferred_element_type=jnp.float32)
    m = jnp.max(cos, axis=1, keepdims=True)
    e = jnp.exp(cos - m)
    alpha = e / jnp.sum(e, axis=1, keepdims=True)
    p_a = lax.dot_general(alpha, p, (((1,), (0,)), ((), ())),
                          preferred_element_type=jnp.float32)
    r = jnp.sum(p, axis=0, keepdims=True) * jnp.float32(1.0 / POOL)
    p_c = p - r
    pa_c = p_a - r
    pk_sq = jnp.sum(p_c * p_c, axis=1, keepdims=True)
    g_t = lax.dot_general(p_c, pa_c, (((1,), (1,)), ((), ())),
                          preferred_element_type=jnp.float32,
                          precision=lax.Precision.HIGHEST)
    scores_t = pk_sq - 2.0 * g_t
    mins = jnp.min(scores_t, axis=0, keepdims=True)
    iota_k = lax.broadcasted_iota(jnp.int32, (POOL, B), 0)
    idx = jnp.min(jnp.where(scores_t == mins, iota_k, POOL),
                  axis=0, keepdims=True)
    idx_ref[...] = idx
    total = (jnp.sum(mins, axis=1, keepdims=True)
             + jnp.sum(pa_c * pa_c, axis=(0, 1), keepdims=True)[:1, :1])
    loss_ref[...] = 0.5 * total / jnp.float32(B * FDIM)


@functools.cache
def _make_sc_gather():
    @functools.partial(
        pl.kernel,
        mesh=plsc.VectorSubcoreMesh(core_axis_name="c", subcore_axis_name="s"),
        out_type=jax.ShapeDtypeStruct((B, FDIM), jnp.float32),
        scratch_types=[
            pltpu.VMEM((ROWS_PER_W,), jnp.int32),
            pltpu.VMEM((ROWS_PER_W, FDIM), jnp.float32),
            pltpu.SemaphoreType.DMA,
        ],
    )
    def _sc_gather(p_hbm, idx_hbm, out_hbm, idx_v, rows_v, sem):
        c = lax.axis_index("c")
        s = lax.axis_index("s")
        wid = s * 2 + c

        @pl.when(wid < NW)
        def _():
            base = wid * ROWS_PER_W
            pltpu.sync_copy(idx_hbm.at[pl.ds(base, ROWS_PER_W)], idx_v)
            pltpu.async_copy(p_hbm.at[idx_v], rows_v, sem).wait()
            pltpu.sync_copy(rows_v, out_hbm.at[pl.ds(base, ROWS_PER_W)])

    return _sc_gather


def kernel(x_querry, l, x_block, e_p_0, e_k_0):
    p_flat = e_p_0.reshape(POOL, FDIM)
    idx2, loss2 = pl.pallas_call(
        _score_body,
        out_shape=(
            jax.ShapeDtypeStruct((1, B), jnp.int32),
            jax.ShapeDtypeStruct((1, 1), jnp.float32),
        ),
    )(x_querry, e_k_0, p_flat)
    quant = _make_sc_gather()(p_flat, idx2.reshape(B))
    q = quant.reshape(B, PLEN, EMB_D)
    Ek = q[:, : PLEN // 2, :]
    Ev = q[:, PLEN // 2:, :]
    loss = loss2[0, 0]
    return (Ek, Ev, loss, x_block)
